# Initial kernel scaffold; baseline (speedup 1.0000x reference)
import jax
import jax.numpy as jnp
from jax.experimental import pallas as pl


def kernel(inputs, embed_weight):
    raise NotImplementedError("write your pallas kernel here")


# speedup vs baseline: 2.2920x; 2.2920x over previous
import jax
import jax.numpy as jnp
from jax.experimental import pallas as pl


def _broadcast_body(w_ref, o_ref):
    o_ref[...] = jnp.broadcast_to(w_ref[...][None, :, :], o_ref.shape)


def kernel(inputs, embed_weight):
    B, S = inputs.shape
    E = embed_weight.shape[1]
    BLK = 512
    n_blocks = pl.cdiv(S, BLK)
    out = pl.pallas_call(
        _broadcast_body,
        grid=(n_blocks,),
        in_specs=[pl.BlockSpec((BLK, E), lambda j: (j, 0))],
        out_specs=pl.BlockSpec((B, BLK, E), lambda j: (0, j, 0)),
        out_shape=jax.ShapeDtypeStruct((B, S, E), embed_weight.dtype),
    )(embed_weight)
    return out


# speedup vs baseline: 2.3592x; 1.0294x over previous
import jax
import jax.numpy as jnp
from jax.experimental import pallas as pl


def _broadcast_body(w_ref, o_ref):
    o_ref[...] = jnp.broadcast_to(w_ref[...][None, :, :], o_ref.shape)


def kernel(inputs, embed_weight):
    B, S = inputs.shape
    E = embed_weight.shape[1]
    BLK = 1024
    n_blocks = pl.cdiv(S, BLK)
    out = pl.pallas_call(
        _broadcast_body,
        grid=(n_blocks,),
        in_specs=[pl.BlockSpec((BLK, E), lambda j: (j, 0))],
        out_specs=pl.BlockSpec((B, BLK, E), lambda j: (0, j, 0)),
        out_shape=jax.ShapeDtypeStruct((B, S, E), embed_weight.dtype),
    )(embed_weight)
    return out
